# Initial kernel scaffold; baseline (speedup 1.0000x reference)
import jax
import jax.numpy as jnp
from jax.experimental import pallas as pl


def kernel(inputs):
    raise NotImplementedError("write your pallas kernel here")


# speedup vs baseline: 20.3265x; 20.3265x over previous
import functools

import jax
import jax.numpy as jnp
from jax.experimental import pallas as pl
from jax.experimental.pallas import tpu as pltpu

_NBINS = 256
_WR = 0.2989
_WG = 0.5870
_WB = 0.1140
_NGRP = 64


def _gray_minmax_kernel(r_ref, g_ref, b_ref, gray_ref, mm_ref, mn_acc, mx_acc,
                        *, k1):
    k = pl.program_id(0)
    p0 = r_ref[...].astype(jnp.bfloat16).astype(jnp.float32) * jnp.float32(
        jnp.bfloat16(_WR))
    p1 = g_ref[...].astype(jnp.bfloat16).astype(jnp.float32) * jnp.float32(
        jnp.bfloat16(_WG))
    p2 = b_ref[...].astype(jnp.bfloat16).astype(jnp.float32) * jnp.float32(
        jnp.bfloat16(_WB))
    gray = (p0 + p1) + p2
    gray_ref[...] = gray

    @pl.when(k == 0)
    def _():
        mn_acc[...] = jnp.full_like(mn_acc, jnp.inf)
        mx_acc[...] = jnp.full_like(mx_acc, -jnp.inf)

    mn_acc[...] = jnp.minimum(mn_acc[...], jnp.min(gray, axis=0, keepdims=True))
    mx_acc[...] = jnp.maximum(mx_acc[...], jnp.max(gray, axis=0, keepdims=True))

    @pl.when(k == k1 - 1)
    def _():
        mnv = jnp.min(mn_acc[...], axis=1, keepdims=True)
        mxv = jnp.max(mx_acc[...], axis=1, keepdims=True)
        mm_ref[0, :, :128] = jnp.broadcast_to(mnv, (1, 128))
        mm_ref[0, :, 128:] = jnp.broadcast_to(mxv, (1, 128))


def _hist_kernel(mn_ref, mx_ref, gray_ref, out_ref, acc_ref, q_scr, byte_scr,
                 *, k2):
    k = pl.program_id(0)

    @pl.when(k == 0)
    def _():
        acc_ref[...] = jnp.zeros_like(acc_ref)

    mn = mn_ref[0, 0]
    mx = mx_ref[0, 0]
    width = (mx - mn) / jnp.float32(_NBINS)
    x = gray_ref[...]
    y = (x - mn) / width
    idx_f = jnp.clip(jnp.floor(y), 0.0, 255.0)
    idx = idx_f.astype(jnp.int32)
    q = jax.lax.shift_right_logical(idx, 2)
    r = jnp.bitwise_and(idx, 3)
    byte = jnp.where(
        r == 0, jnp.int32(1),
        jnp.where(r == 1, jnp.int32(1 << 8),
                  jnp.where(r == 2, jnp.int32(1 << 16), jnp.int32(1 << 24))))
    q_scr[...] = q
    byte_scr[...] = byte

    def body(gi, _):
        qs = q_scr[...]
        bs = byte_scr[...]
        acc_ref[gi] = acc_ref[gi] + jnp.where(qs == gi, bs, 0)
        return ()

    jax.lax.fori_loop(0, _NGRP, body, (), unroll=2)

    @pl.when(k == k2 - 1)
    def _():
        for byte_pos in range(4):
            v = jnp.bitwise_and(
                jax.lax.shift_right_logical(acc_ref[...], 8 * byte_pos), 255)
            out_ref[0, byte_pos] = jnp.sum(v.astype(jnp.float32), axis=1)


def _otsu_binarize_kernel(mn_ref, mx_ref, counts_ref, gray_ref, out_ref,
                          thr_ref):
    k = pl.program_id(0)

    @pl.when(k == 0)
    def _():
        mn = mn_ref[0, 0]
        mx = mx_ref[0, 0]
        width = (mx - mn) / jnp.float32(_NBINS)
        c = counts_ref[...]
        lane = jax.lax.broadcasted_iota(
            jnp.int32, (1, _NBINS), 1).astype(jnp.float32)
        centers = mn + width * (lane + 0.5)
        prods = c * centers
        ri = jax.lax.broadcasted_iota(jnp.int32, (_NBINS, _NBINS), 0)
        ci = jax.lax.broadcasted_iota(jnp.int32, (_NBINS, _NBINS), 1)
        tri = jnp.where(ri <= ci, jnp.float32(1.0), jnp.float32(0.0))
        w1 = jax.lax.dot(c, tri, precision=jax.lax.Precision.HIGHEST,
                         preferred_element_type=jnp.float32)
        s1 = jax.lax.dot(prods, tri, precision=jax.lax.Precision.HIGHEST,
                         preferred_element_type=jnp.float32)
        ntot = w1[0, _NBINS - 1]
        stot = s1[0, _NBINS - 1]
        w2 = ntot - w1 + c
        s2 = stot - s1 + prods
        m1 = s1 / jnp.maximum(w1, 1.0)
        m2 = s2 / jnp.maximum(w2, 1.0)
        w2s = jnp.concatenate([w2[:, 1:], w2[:, :1]], axis=1)
        m2s = jnp.concatenate([m2[:, 1:], m2[:, :1]], axis=1)
        d = m1 - m2s
        score = w1 * w2s * d * d
        valid = jax.lax.broadcasted_iota(jnp.int32, (1, _NBINS), 1) < (_NBINS - 1)
        score = jnp.where(valid, score, jnp.float32(-1.0))
        best = jnp.max(score, axis=1, keepdims=True)
        thr = jnp.min(jnp.where(score == best, centers, jnp.inf),
                      axis=1, keepdims=True)
        thr_ref[0, 0] = thr[0, 0]

    t = thr_ref[0, 0]
    out_ref[...] = jnp.where(gray_ref[...] > t, jnp.float32(1.0),
                             jnp.float32(0.0))


def kernel(inputs):
    bsz, hh, ww, _ = inputs.shape
    rows = bsz * hh
    cols = ww
    br1 = min(512, rows)
    k1 = rows // br1
    br2 = min(128, rows)
    k2 = rows // br2

    x = inputs[..., :3]
    r = x[..., 0].reshape(rows, cols)
    g = x[..., 1].reshape(rows, cols)
    b = x[..., 2].reshape(rows, cols)

    cparams = pltpu.CompilerParams(
        dimension_semantics=("arbitrary",),
        vmem_limit_bytes=100 * 1024 * 1024,
    )

    row_spec1 = pl.BlockSpec((br1, cols), lambda k: (k, 0))
    gray, mm = pl.pallas_call(
        functools.partial(_gray_minmax_kernel, k1=k1),
        out_shape=(
            jax.ShapeDtypeStruct((rows, cols), jnp.float32),
            jax.ShapeDtypeStruct((1, 1, 256), jnp.float32),
        ),
        grid=(k1,),
        in_specs=[row_spec1, row_spec1, row_spec1],
        out_specs=(
            row_spec1,
            pl.BlockSpec((1, 1, 256), lambda k: (0, 0, 0)),
        ),
        scratch_shapes=[
            pltpu.VMEM((1, cols), jnp.float32),
            pltpu.VMEM((1, cols), jnp.float32),
        ],
        compiler_params=cparams,
        name="gray_minmax",
    )(r, g, b)

    mn = jnp.min(mm[:, 0, 0]).reshape(1, 1)
    mx = jnp.max(mm[:, 0, 128]).reshape(1, 1)

    smem_spec = pl.BlockSpec(memory_space=pltpu.SMEM)
    row_spec2 = pl.BlockSpec((br2, cols), lambda k: (k, 0))
    hist = pl.pallas_call(
        functools.partial(_hist_kernel, k2=k2),
        out_shape=jax.ShapeDtypeStruct((1, 4, _NGRP, cols), jnp.float32),
        grid=(k2,),
        in_specs=[smem_spec, smem_spec, row_spec2],
        out_specs=pl.BlockSpec((1, 4, _NGRP, cols), lambda k: (0, 0, 0, 0)),
        scratch_shapes=[
            pltpu.VMEM((_NGRP, br2, cols), jnp.int32),
            pltpu.VMEM((br2, cols), jnp.int32),
            pltpu.VMEM((br2, cols), jnp.int32),
        ],
        compiler_params=cparams,
        name="hist256",
    )(mn, mx, gray)

    counts = hist.sum(axis=(0, 3)).T.reshape(1, _NBINS)

    out = pl.pallas_call(
        _otsu_binarize_kernel,
        out_shape=jax.ShapeDtypeStruct((rows, cols), jnp.float32),
        grid=(k1,),
        in_specs=[
            smem_spec,
            smem_spec,
            pl.BlockSpec((1, _NBINS), lambda k: (0, 0)),
            row_spec1,
        ],
        out_specs=row_spec1,
        scratch_shapes=[pltpu.SMEM((1, 1), jnp.float32)],
        compiler_params=cparams,
        name="otsu_binarize",
    )(mn, mx, counts, gray)

    return out.reshape(bsz, hh, ww, 1)
